# Initial kernel scaffold; baseline (speedup 1.0000x reference)
import jax
import jax.numpy as jnp
from jax.experimental import pallas as pl


def kernel(token_ids, mu_table, logvar_table):
    raise NotImplementedError("write your pallas kernel here")


# speedup vs baseline: 1.0398x; 1.0398x over previous
import functools

import jax
import jax.numpy as jnp
from jax import lax
from jax.experimental import pallas as pl
from jax.experimental.pallas import tpu as pltpu
from jax.experimental.pallas import tpu_sc as plsc

BATCH = 4096
HIST = 50
D = 64
B = BATCH * HIST
NW = 32
BPW = B // NW
C = 320
NCHUNK = BPW // C


def _embed_body(ids_hbm, mu_hbm, lv_hbm, mu_out, var_out,
                idx_v, mu_v, lv_v, sem_mu, sem_lv):
    wid = lax.axis_index("s") * 2 + lax.axis_index("c")
    base = wid * BPW

    def chunk_body(ci, carry):
        off = base + ci * C
        pltpu.sync_copy(ids_hbm.at[pl.ds(off, C)], idx_v)
        cp_mu = pltpu.async_copy(mu_hbm.at[idx_v], mu_v, sem_mu)
        cp_lv = pltpu.async_copy(lv_hbm.at[idx_v], lv_v, sem_lv)
        cp_mu.wait()
        pltpu.sync_copy(mu_v, mu_out.at[pl.ds(off, C)])
        cp_lv.wait()

        def erow(r, carry2):
            for c4 in range(D // 16):
                sl = pl.ds(c4 * 16, 16)
                lv_v[r, sl] = jnp.exp(lv_v[r, sl])
            return carry2

        lax.fori_loop(0, C, erow, 0)
        pltpu.sync_copy(lv_v, var_out.at[pl.ds(off, C)])
        return carry

    lax.fori_loop(0, NCHUNK, chunk_body, 0)


@jax.jit
def _embed(ids_flat, mu_table, logvar_table):
    mesh = plsc.VectorSubcoreMesh(core_axis_name="c", subcore_axis_name="s")
    f = functools.partial(
        pl.kernel,
        mesh=mesh,
        compiler_params=pltpu.CompilerParams(use_tc_tiling_on_sc=False),
        out_type=(
            jax.ShapeDtypeStruct((B, D), jnp.float32),
            jax.ShapeDtypeStruct((B, D), jnp.float32),
        ),
        scratch_types=[
            pltpu.VMEM((C,), jnp.int32),
            pltpu.VMEM((C, D), jnp.float32),
            pltpu.VMEM((C, D), jnp.float32),
            pltpu.SemaphoreType.DMA,
            pltpu.SemaphoreType.DMA,
        ],
    )(_embed_body)
    return f(ids_flat, mu_table, logvar_table)


def kernel(token_ids, mu_table, logvar_table):
    ids_flat = token_ids.reshape(B).astype(jnp.int32)
    mu, var = _embed(ids_flat, mu_table, logvar_table)
    return (mu.reshape(BATCH, HIST, D), var.reshape(BATCH, HIST, D))


# speedup vs baseline: 1.0952x; 1.0532x over previous
import functools

import jax
import jax.numpy as jnp
from jax import lax
from jax.experimental import pallas as pl
from jax.experimental.pallas import tpu as pltpu
from jax.experimental.pallas import tpu_sc as plsc

BATCH = 4096
HIST = 50
D = 64
B = BATCH * HIST
NW = 32
BPW = B // NW
C = 320
NCHUNK = BPW // C


def _embed_body(ids_hbm, mu_hbm, lv_hbm, mu_out, var_out,
                idx0, idx1, mu0, mu1, lv0, lv1,
                sg0, sg1, sw0, sw1):
    wid = lax.axis_index("s") * 2 + lax.axis_index("c")
    base = wid * BPW
    idx_v = (idx0, idx1)
    mu_v = (mu0, mu1)
    lv_v = (lv0, lv1)
    sem_g = (sg0, sg1)
    sem_w = (sw0, sw1)

    pend_g = [None, None]
    pend_w = [None, None]

    def issue_gather(ci, b):
        off = base + ci * C
        pltpu.sync_copy(ids_hbm.at[pl.ds(off, C)], idx_v[b])
        cp_mu = pltpu.async_copy(mu_hbm.at[idx_v[b]], mu_v[b], sem_g[b])
        cp_lv = pltpu.async_copy(lv_hbm.at[idx_v[b]], lv_v[b], sem_g[b])
        pend_g[b] = (cp_mu, cp_lv)

    issue_gather(0, 0)
    for ci in range(NCHUNK):
        b = ci & 1
        nb = 1 - b
        if ci + 1 < NCHUNK:
            if pend_w[nb] is not None:
                for cp in pend_w[nb]:
                    cp.wait()
            issue_gather(ci + 1, nb)
        off = base + ci * C
        cp_mu, cp_lv = pend_g[b]
        cp_mu.wait()
        w_mu = pltpu.async_copy(mu_v[b], mu_out.at[pl.ds(off, C)], sem_w[b])
        cp_lv.wait()

        lv = lv_v[b]

        @plsc.parallel_loop(0, C, unroll=8)
        def erow(r):
            for c4 in range(D // 16):
                sl = pl.ds(c4 * 16, 16)
                lv[r, sl] = jnp.exp(lv[r, sl])

        w_lv = pltpu.async_copy(lv, var_out.at[pl.ds(off, C)], sem_w[b])
        pend_w[b] = (w_mu, w_lv)

    for b in (0, 1):
        for cp in pend_w[b]:
            cp.wait()


@jax.jit
def _embed(ids_flat, mu_table, logvar_table):
    mesh = plsc.VectorSubcoreMesh(core_axis_name="c", subcore_axis_name="s")
    f = functools.partial(
        pl.kernel,
        mesh=mesh,
        compiler_params=pltpu.CompilerParams(use_tc_tiling_on_sc=False),
        out_type=(
            jax.ShapeDtypeStruct((B, D), jnp.float32),
            jax.ShapeDtypeStruct((B, D), jnp.float32),
        ),
        scratch_types=[
            pltpu.VMEM((C,), jnp.int32),
            pltpu.VMEM((C,), jnp.int32),
            pltpu.VMEM((C, D), jnp.float32),
            pltpu.VMEM((C, D), jnp.float32),
            pltpu.VMEM((C, D), jnp.float32),
            pltpu.VMEM((C, D), jnp.float32),
            pltpu.SemaphoreType.DMA,
            pltpu.SemaphoreType.DMA,
            pltpu.SemaphoreType.DMA,
            pltpu.SemaphoreType.DMA,
        ],
    )(_embed_body)
    return f(ids_flat, mu_table, logvar_table)


def kernel(token_ids, mu_table, logvar_table):
    ids_flat = token_ids.reshape(B).astype(jnp.int32)
    mu, var = _embed(ids_flat, mu_table, logvar_table)
    return (mu.reshape(BATCH, HIST, D), var.reshape(BATCH, HIST, D))


# speedup vs baseline: 1.5993x; 1.4603x over previous
import functools

import jax
import jax.numpy as jnp
from jax import lax
from jax.experimental import pallas as pl
from jax.experimental.pallas import tpu as pltpu
from jax.experimental.pallas import tpu_sc as plsc

VOCAB = 1000000
BATCH = 4096
HIST = 50
D = 64
B = BATCH * HIST
NW = 32
BPW = B // NW
C = 320
NCHUNK = BPW // C

NB = 2048
GRID = (VOCAB + NB - 1) // NB
VLIN = GRID * NB


def _fmt_body(mu_ref, lv_ref, mu_out, lv_out):
    mu = mu_ref[...]
    lv = lv_ref[...]
    mu_out[:, 0:64] = mu[:, : NB // 2].T
    mu_out[:, 64:128] = mu[:, NB // 2 :].T
    lv_out[:, 0:64] = jnp.exp(lv[:, : NB // 2].T)
    lv_out[:, 64:128] = jnp.exp(lv[:, NB // 2 :].T)


def _tc_format(mu_t, lv_t):
    return pl.pallas_call(
        _fmt_body,
        grid=(GRID,),
        in_specs=[
            pl.BlockSpec((64, NB), lambda i: (0, i)),
            pl.BlockSpec((64, NB), lambda i: (0, i)),
        ],
        out_specs=[
            pl.BlockSpec((NB // 2, 128), lambda i: (i, 0)),
            pl.BlockSpec((NB // 2, 128), lambda i: (i, 0)),
        ],
        out_shape=[
            jax.ShapeDtypeStruct((VLIN // 2, 128), jnp.float32),
            jax.ShapeDtypeStruct((VLIN // 2, 128), jnp.float32),
        ],
    )(mu_t, lv_t)


def _gather_body(ids_hbm, mu_hbm, var_hbm, mu_out, var_out,
                 idx0, idx1, mu0, mu1, lv0, lv1,
                 sg0, sg1, sw0, sw1):
    wid = lax.axis_index("s") * 2 + lax.axis_index("c")
    base = wid * BPW
    idx_v = (idx0, idx1)
    mu_v = (mu0, mu1)
    lv_v = (lv0, lv1)
    sem_g = (sg0, sg1)
    sem_w = (sw0, sw1)

    pend_g = [None, None]
    pend_w = [None, None]

    def issue_gather(ci, b):
        off = base + ci * C
        pltpu.sync_copy(ids_hbm.at[pl.ds(off, C)], idx_v[b])
        cp_mu = pltpu.async_copy(mu_hbm.at[idx_v[b]], mu_v[b], sem_g[b])
        cp_lv = pltpu.async_copy(var_hbm.at[idx_v[b]], lv_v[b], sem_g[b])
        pend_g[b] = (cp_mu, cp_lv)

    issue_gather(0, 0)
    for ci in range(NCHUNK):
        b = ci & 1
        nb = 1 - b
        if ci + 1 < NCHUNK:
            if pend_w[nb] is not None:
                for cp in pend_w[nb]:
                    cp.wait()
            issue_gather(ci + 1, nb)
        off = base + ci * C
        cp_mu, cp_lv = pend_g[b]
        cp_mu.wait()
        w_mu = pltpu.async_copy(mu_v[b], mu_out.at[pl.ds(off, C)], sem_w[b])
        cp_lv.wait()
        w_lv = pltpu.async_copy(lv_v[b], var_out.at[pl.ds(off, C)], sem_w[b])
        pend_w[b] = (w_mu, w_lv)

    for b in (0, 1):
        for cp in pend_w[b]:
            cp.wait()


def _sc_gather(ids_flat, mu_lin, var_lin):
    mesh = plsc.VectorSubcoreMesh(core_axis_name="c", subcore_axis_name="s")
    f = functools.partial(
        pl.kernel,
        mesh=mesh,
        compiler_params=pltpu.CompilerParams(use_tc_tiling_on_sc=False),
        out_type=(
            jax.ShapeDtypeStruct((B, D), jnp.float32),
            jax.ShapeDtypeStruct((B, D), jnp.float32),
        ),
        scratch_types=[
            pltpu.VMEM((C,), jnp.int32),
            pltpu.VMEM((C,), jnp.int32),
            pltpu.VMEM((C, D), jnp.float32),
            pltpu.VMEM((C, D), jnp.float32),
            pltpu.VMEM((C, D), jnp.float32),
            pltpu.VMEM((C, D), jnp.float32),
            pltpu.SemaphoreType.DMA,
            pltpu.SemaphoreType.DMA,
            pltpu.SemaphoreType.DMA,
            pltpu.SemaphoreType.DMA,
        ],
    )(_gather_body)
    return f(ids_flat, mu_lin, var_lin)


@jax.jit
def _pipeline(token_ids, mu_table, logvar_table):
    mu128, var128 = _tc_format(mu_table.T, logvar_table.T)
    mu_lin = mu128.reshape(VLIN, D)
    var_lin = var128.reshape(VLIN, D)
    ids = token_ids.reshape(B).astype(jnp.int32)
    l = ids & (NB - 1)
    ids2 = (ids - l) + jnp.where(l < NB // 2, 2 * l, 2 * l - (NB - 1))
    mu, var = _sc_gather(ids2, mu_lin, var_lin)
    return (mu.reshape(BATCH, HIST, D), var.reshape(BATCH, HIST, D))


def kernel(token_ids, mu_table, logvar_table):
    return _pipeline(token_ids, mu_table, logvar_table)


# speedup vs baseline: 1.9200x; 1.2005x over previous
import functools

import jax
import jax.numpy as jnp
from jax import lax
from jax.experimental import pallas as pl
from jax.experimental.pallas import tpu as pltpu
from jax.experimental.pallas import tpu_sc as plsc

VOCAB = 1000000
BATCH = 4096
HIST = 50
D = 64
B = BATCH * HIST
NW = 32
BPW = B // NW
C = 320
NCHUNK = BPW // C

NB = 4096
GRID = (VOCAB + NB - 1) // NB
VLIN = GRID * NB


def _fmt_body(mu_ref, lv_ref, mu_out, lv_out):
    mu = mu_ref[...]
    lv = lv_ref[...]
    mu_out[:, 0:64] = mu[:, : NB // 2].T
    mu_out[:, 64:128] = mu[:, NB // 2 :].T
    lv_out[:, 0:64] = jnp.exp(lv[:, : NB // 2].T)
    lv_out[:, 64:128] = jnp.exp(lv[:, NB // 2 :].T)


def _tc_format(mu_t, lv_t):
    return pl.pallas_call(
        _fmt_body,
        grid=(GRID,),
        in_specs=[
            pl.BlockSpec((64, NB), lambda i: (0, i)),
            pl.BlockSpec((64, NB), lambda i: (0, i)),
        ],
        out_specs=[
            pl.BlockSpec((NB // 2, 128), lambda i: (i, 0)),
            pl.BlockSpec((NB // 2, 128), lambda i: (i, 0)),
        ],
        out_shape=[
            jax.ShapeDtypeStruct((VLIN // 2, 128), jnp.float32),
            jax.ShapeDtypeStruct((VLIN // 2, 128), jnp.float32),
        ],
    )(mu_t, lv_t)


def _gather_body(ids_hbm, mu_hbm, var_hbm, mu_out, var_out,
                 idx0, idx1, mu0, mu1, lv0, lv1,
                 sg0, sg1, sw0, sw1):
    wid = lax.axis_index("s") * 2 + lax.axis_index("c")
    base = wid * BPW
    idx_v = (idx0, idx1)
    mu_v = (mu0, mu1)
    lv_v = (lv0, lv1)
    sem_g = (sg0, sg1)
    sem_w = (sw0, sw1)

    pend_g = [None, None]
    pend_w = [None, None]

    def issue_gather(ci, b):
        off = base + ci * C
        pltpu.sync_copy(ids_hbm.at[pl.ds(off, C)], idx_v[b])
        cp_mu = pltpu.async_copy(mu_hbm.at[idx_v[b]], mu_v[b], sem_g[b])
        cp_lv = pltpu.async_copy(var_hbm.at[idx_v[b]], lv_v[b], sem_g[b])
        pend_g[b] = (cp_mu, cp_lv)

    issue_gather(0, 0)
    for ci in range(NCHUNK):
        b = ci & 1
        nb = 1 - b
        if ci + 1 < NCHUNK:
            if pend_w[nb] is not None:
                for cp in pend_w[nb]:
                    cp.wait()
            issue_gather(ci + 1, nb)
        off = base + ci * C
        cp_mu, cp_lv = pend_g[b]
        cp_mu.wait()
        w_mu = pltpu.async_copy(mu_v[b], mu_out.at[pl.ds(off, C)], sem_w[b])
        cp_lv.wait()
        w_lv = pltpu.async_copy(lv_v[b], var_out.at[pl.ds(off, C)], sem_w[b])
        pend_w[b] = (w_mu, w_lv)

    for b in (0, 1):
        for cp in pend_w[b]:
            cp.wait()


def _sc_gather(ids_flat, mu_lin, var_lin):
    mesh = plsc.VectorSubcoreMesh(core_axis_name="c", subcore_axis_name="s")
    f = functools.partial(
        pl.kernel,
        mesh=mesh,
        compiler_params=pltpu.CompilerParams(use_tc_tiling_on_sc=False),
        out_type=(
            jax.ShapeDtypeStruct((B, D), jnp.float32),
            jax.ShapeDtypeStruct((B, D), jnp.float32),
        ),
        scratch_types=[
            pltpu.VMEM((C,), jnp.int32),
            pltpu.VMEM((C,), jnp.int32),
            pltpu.VMEM((C, D), jnp.float32),
            pltpu.VMEM((C, D), jnp.float32),
            pltpu.VMEM((C, D), jnp.float32),
            pltpu.VMEM((C, D), jnp.float32),
            pltpu.SemaphoreType.DMA,
            pltpu.SemaphoreType.DMA,
            pltpu.SemaphoreType.DMA,
            pltpu.SemaphoreType.DMA,
        ],
    )(_gather_body)
    return f(ids_flat, mu_lin, var_lin)


@jax.jit
def _pipeline(token_ids, mu_table, logvar_table):
    mu128, var128 = _tc_format(mu_table.T, logvar_table.T)
    mu_lin = mu128.reshape(VLIN, D)
    var_lin = var128.reshape(VLIN, D)
    ids = token_ids.T.reshape(B).astype(jnp.int32)
    l = ids & (NB - 1)
    ids2 = (ids - l) + jnp.where(l < NB // 2, 2 * l, 2 * l - (NB - 1))
    mu, var = _sc_gather(ids2, mu_lin, var_lin)
    mu = mu.reshape(HIST, BATCH, D).transpose(1, 0, 2)
    var = var.reshape(HIST, BATCH, D).transpose(1, 0, 2)
    return (mu, var)


def kernel(token_ids, mu_table, logvar_table):
    return _pipeline(token_ids, mu_table, logvar_table)


# speedup vs baseline: 2.1303x; 1.1095x over previous
import functools

import jax
import jax.numpy as jnp
from jax import lax
from jax.experimental import pallas as pl
from jax.experimental.pallas import tpu as pltpu
from jax.experimental.pallas import tpu_sc as plsc

VOCAB = 1000000
BATCH = 4096
HIST = 50
D = 64
B = BATCH * HIST
NW = 32
BPW = B // NW
C = 320
NCHUNK = BPW // C

NB = 8192
GRID = (VOCAB + NB - 1) // NB
VLIN = GRID * NB


def _fmt_body(mu_ref, lv_ref, mu_out, lv_out):
    mu = mu_ref[...]
    lv = lv_ref[...]
    mu_out[:, 0:64] = mu[:, : NB // 2].T
    mu_out[:, 64:128] = mu[:, NB // 2 :].T
    lv_out[:, 0:64] = jnp.exp(lv[:, : NB // 2].T)
    lv_out[:, 64:128] = jnp.exp(lv[:, NB // 2 :].T)


def _tc_format(mu_t, lv_t):
    return pl.pallas_call(
        _fmt_body,
        grid=(GRID,),
        in_specs=[
            pl.BlockSpec((64, NB), lambda i: (0, i)),
            pl.BlockSpec((64, NB), lambda i: (0, i)),
        ],
        out_specs=[
            pl.BlockSpec((NB // 2, 128), lambda i: (i, 0)),
            pl.BlockSpec((NB // 2, 128), lambda i: (i, 0)),
        ],
        out_shape=[
            jax.ShapeDtypeStruct((VLIN // 2, 128), jnp.float32),
            jax.ShapeDtypeStruct((VLIN // 2, 128), jnp.float32),
        ],
    )(mu_t, lv_t)


def _gather_body(ids_hbm, mu_hbm, var_hbm, mu_out, var_out,
                 idx0, idx1, mu0, mu1, lv0, lv1,
                 sg0, sg1, sw0, sw1):
    wid = lax.axis_index("s") * 2 + lax.axis_index("c")
    base = wid * BPW
    idx_v = (idx0, idx1)
    mu_v = (mu0, mu1)
    lv_v = (lv0, lv1)
    sem_g = (sg0, sg1)
    sem_w = (sw0, sw1)

    pend_g = [None, None]
    pend_w = [None, None]

    def issue_gather(ci, b):
        off = base + ci * C
        pltpu.sync_copy(ids_hbm.at[pl.ds(off, C)], idx_v[b])
        cp_mu = pltpu.async_copy(mu_hbm.at[idx_v[b]], mu_v[b], sem_g[b])
        cp_lv = pltpu.async_copy(var_hbm.at[idx_v[b]], lv_v[b], sem_g[b])
        pend_g[b] = (cp_mu, cp_lv)

    issue_gather(0, 0)
    for ci in range(NCHUNK):
        b = ci & 1
        nb = 1 - b
        if ci + 1 < NCHUNK:
            if pend_w[nb] is not None:
                for cp in pend_w[nb]:
                    cp.wait()
            issue_gather(ci + 1, nb)
        off = base + ci * C
        cp_mu, cp_lv = pend_g[b]
        cp_mu.wait()
        w_mu = pltpu.async_copy(mu_v[b], mu_out.at[pl.ds(off, C)], sem_w[b])
        cp_lv.wait()
        w_lv = pltpu.async_copy(lv_v[b], var_out.at[pl.ds(off, C)], sem_w[b])
        pend_w[b] = (w_mu, w_lv)

    for b in (0, 1):
        for cp in pend_w[b]:
            cp.wait()


def _sc_gather(ids_flat, mu_lin, var_lin):
    mesh = plsc.VectorSubcoreMesh(core_axis_name="c", subcore_axis_name="s")
    f = functools.partial(
        pl.kernel,
        mesh=mesh,
        compiler_params=pltpu.CompilerParams(use_tc_tiling_on_sc=False),
        out_type=(
            jax.ShapeDtypeStruct((B, D), jnp.float32),
            jax.ShapeDtypeStruct((B, D), jnp.float32),
        ),
        scratch_types=[
            pltpu.VMEM((C,), jnp.int32),
            pltpu.VMEM((C,), jnp.int32),
            pltpu.VMEM((C, D), jnp.float32),
            pltpu.VMEM((C, D), jnp.float32),
            pltpu.VMEM((C, D), jnp.float32),
            pltpu.VMEM((C, D), jnp.float32),
            pltpu.SemaphoreType.DMA,
            pltpu.SemaphoreType.DMA,
            pltpu.SemaphoreType.DMA,
            pltpu.SemaphoreType.DMA,
        ],
    )(_gather_body)
    return f(ids_flat, mu_lin, var_lin)


@jax.jit
def _pipeline(token_ids, mu_table, logvar_table):
    mu128, var128 = _tc_format(mu_table.T, logvar_table.T)
    mu_lin = mu128.reshape(VLIN, D)
    var_lin = var128.reshape(VLIN, D)
    ids = token_ids.T.reshape(B).astype(jnp.int32)
    l = ids & (NB - 1)
    ids2 = (ids - l) + jnp.where(l < NB // 2, 2 * l, 2 * l - (NB - 1))
    mu, var = _sc_gather(ids2, mu_lin, var_lin)
    mu = mu.reshape(HIST, BATCH, D).transpose(1, 0, 2)
    var = var.reshape(HIST, BATCH, D).transpose(1, 0, 2)
    return (mu, var)


def kernel(token_ids, mu_table, logvar_table):
    return _pipeline(token_ids, mu_table, logvar_table)


# speedup vs baseline: 2.1483x; 1.0085x over previous
import functools

import jax
import jax.numpy as jnp
from jax import lax
from jax.experimental import pallas as pl
from jax.experimental.pallas import tpu as pltpu
from jax.experimental.pallas import tpu_sc as plsc

VOCAB = 1000000
BATCH = 4096
HIST = 50
D = 64
B = BATCH * HIST
NW = 32
BPW = B // NW
C = 320
NCHUNK = BPW // C

NB = 16384
GRID = (VOCAB + NB - 1) // NB
VLIN = GRID * NB


def _fmt_body(mu_ref, lv_ref, mu_out, lv_out):
    mu = mu_ref[...]
    lv = lv_ref[...]
    mu_out[:, 0:64] = mu[:, : NB // 2].T
    mu_out[:, 64:128] = mu[:, NB // 2 :].T
    lv_out[:, 0:64] = jnp.exp(lv[:, : NB // 2].T)
    lv_out[:, 64:128] = jnp.exp(lv[:, NB // 2 :].T)


def _tc_format(mu_t, lv_t):
    return pl.pallas_call(
        _fmt_body,
        grid=(GRID,),
        in_specs=[
            pl.BlockSpec((64, NB), lambda i: (0, i)),
            pl.BlockSpec((64, NB), lambda i: (0, i)),
        ],
        out_specs=[
            pl.BlockSpec((NB // 2, 128), lambda i: (i, 0)),
            pl.BlockSpec((NB // 2, 128), lambda i: (i, 0)),
        ],
        out_shape=[
            jax.ShapeDtypeStruct((VLIN // 2, 128), jnp.float32),
            jax.ShapeDtypeStruct((VLIN // 2, 128), jnp.float32),
        ],
    )(mu_t, lv_t)


def _gather_body(ids_hbm, mu_hbm, var_hbm, mu_out, var_out,
                 idx0, idx1, mu0, mu1, lv0, lv1,
                 sg0, sg1, sw0, sw1):
    wid = lax.axis_index("s") * 2 + lax.axis_index("c")
    base = wid * BPW
    idx_v = (idx0, idx1)
    mu_v = (mu0, mu1)
    lv_v = (lv0, lv1)
    sem_g = (sg0, sg1)
    sem_w = (sw0, sw1)

    pend_g = [None, None]
    pend_w = [None, None]

    def issue_gather(ci, b):
        off = base + ci * C
        pltpu.sync_copy(ids_hbm.at[pl.ds(off, C)], idx_v[b])
        cp_mu = pltpu.async_copy(mu_hbm.at[idx_v[b]], mu_v[b], sem_g[b])
        cp_lv = pltpu.async_copy(var_hbm.at[idx_v[b]], lv_v[b], sem_g[b])
        pend_g[b] = (cp_mu, cp_lv)

    issue_gather(0, 0)
    for ci in range(NCHUNK):
        b = ci & 1
        nb = 1 - b
        if ci + 1 < NCHUNK:
            if pend_w[nb] is not None:
                for cp in pend_w[nb]:
                    cp.wait()
            issue_gather(ci + 1, nb)
        off = base + ci * C
        cp_mu, cp_lv = pend_g[b]
        cp_mu.wait()
        w_mu = pltpu.async_copy(mu_v[b], mu_out.at[pl.ds(off, C)], sem_w[b])
        cp_lv.wait()
        w_lv = pltpu.async_copy(lv_v[b], var_out.at[pl.ds(off, C)], sem_w[b])
        pend_w[b] = (w_mu, w_lv)

    for b in (0, 1):
        for cp in pend_w[b]:
            cp.wait()


def _sc_gather(ids_flat, mu_lin, var_lin):
    mesh = plsc.VectorSubcoreMesh(core_axis_name="c", subcore_axis_name="s")
    f = functools.partial(
        pl.kernel,
        mesh=mesh,
        compiler_params=pltpu.CompilerParams(use_tc_tiling_on_sc=False),
        out_type=(
            jax.ShapeDtypeStruct((B, D), jnp.float32),
            jax.ShapeDtypeStruct((B, D), jnp.float32),
        ),
        scratch_types=[
            pltpu.VMEM((C,), jnp.int32),
            pltpu.VMEM((C,), jnp.int32),
            pltpu.VMEM((C, D), jnp.float32),
            pltpu.VMEM((C, D), jnp.float32),
            pltpu.VMEM((C, D), jnp.float32),
            pltpu.VMEM((C, D), jnp.float32),
            pltpu.SemaphoreType.DMA,
            pltpu.SemaphoreType.DMA,
            pltpu.SemaphoreType.DMA,
            pltpu.SemaphoreType.DMA,
        ],
    )(_gather_body)
    return f(ids_flat, mu_lin, var_lin)


@jax.jit
def _pipeline(token_ids, mu_table, logvar_table):
    mu128, var128 = _tc_format(mu_table.T, logvar_table.T)
    mu_lin = mu128.reshape(VLIN, D)
    var_lin = var128.reshape(VLIN, D)
    ids = token_ids.T.reshape(B).astype(jnp.int32)
    l = ids & (NB - 1)
    ids2 = (ids - l) + jnp.where(l < NB // 2, 2 * l, 2 * l - (NB - 1))
    mu, var = _sc_gather(ids2, mu_lin, var_lin)
    mu = mu.reshape(HIST, BATCH, D).transpose(1, 0, 2)
    var = var.reshape(HIST, BATCH, D).transpose(1, 0, 2)
    return (mu, var)


def kernel(token_ids, mu_table, logvar_table):
    return _pipeline(token_ids, mu_table, logvar_table)
